# baseline (speedup 1.0000x reference)
import functools
import jax
import jax.numpy as jnp
from jax.experimental import pallas as pl

N_BUS = 50000
N_EDGE = 800000
LATENT = 10
HIDDEN = 10
K = 5
GAMMA = 0.9

EDGE_BLK = 8000
BUS_BLK = 5000

_SLOPE = 0.01


def _leaky(x):
    return jnp.where(x >= 0, x, _SLOPE * x)


def _edge_mlp_body(x_ref, w1_ref, b1_ref, w2_ref, b2_ref, w4_ref, b4_ref,
                   o_ref):
    x = x_ref[...]
    h = _leaky(jnp.dot(x, w1_ref[...], preferred_element_type=jnp.float32)
               + b1_ref[...])
    h = _leaky(jnp.dot(h, w2_ref[...], preferred_element_type=jnp.float32)
               + b2_ref[...])
    o_ref[...] = (jnp.dot(h, w4_ref[...], preferred_element_type=jnp.float32)
                  + b4_ref[...])


def _edge_mlp(x, w1, b1, w2, b2, w4, b4):
    n = x.shape[0]
    grid = n // EDGE_BLK
    full = lambda s: pl.BlockSpec(s, lambda i: (0, 0))
    return pl.pallas_call(
        _edge_mlp_body,
        grid=(grid,),
        in_specs=[
            pl.BlockSpec((EDGE_BLK, x.shape[1]), lambda i: (i, 0)),
            full(w1.shape), full(b1.shape), full(w2.shape), full(b2.shape),
            full(w4.shape), full(b4.shape),
        ],
        out_specs=pl.BlockSpec((EDGE_BLK, 3), lambda i: (i, 0)),
        out_shape=jax.ShapeDtypeStruct((n, 3), jnp.float32),
    )(x, w1, b1, w2, b2, w4, b4)


def _bus_mlp_body(x_ref, w1_ref, b1_ref, w2_ref, b2_ref, w4_ref, b4_ref,
                  o_ref):
    x = x_ref[...]
    h = _leaky(jnp.dot(x, w1_ref[...], preferred_element_type=jnp.float32)
               + b1_ref[...])
    h = _leaky(jnp.dot(h, w2_ref[...], preferred_element_type=jnp.float32)
               + b2_ref[...])
    y = (jnp.dot(h, w4_ref[...], preferred_element_type=jnp.float32)
         + b4_ref[...])
    o_ref[...] = jnp.concatenate(
        [y[:, 0:1] + x[:, 1:2], y[:, 1:2] + x[:, 0:1], y[:, 2:12]], axis=1)


def _bus_mlp(x, w1, b1, w2, b2, w4, b4):
    n = x.shape[0]
    grid = n // BUS_BLK
    full = lambda s: pl.BlockSpec(s, lambda i: (0, 0))
    return pl.pallas_call(
        _bus_mlp_body,
        grid=(grid,),
        in_specs=[
            pl.BlockSpec((BUS_BLK, x.shape[1]), lambda i: (i, 0)),
            full(w1.shape), full(b1.shape), full(w2.shape), full(b2.shape),
            full(w4.shape), full(b4.shape),
        ],
        out_specs=pl.BlockSpec((BUS_BLK, 12), lambda i: (i, 0)),
        out_shape=jax.ShapeDtypeStruct((n, 12), jnp.float32),
    )(x, w1, b1, w2, b2, w4, b4)


PHY_R = 1280
PHY_C = 640
PHY_RBLK = 128
PHY_N = PHY_R * PHY_C


def _physics_body(g_ref, o_ref, acc_ref):
    i = pl.program_id(0)
    g = g_ref[...]
    th_s = g[0]
    th_d = g[1]
    v_s = g[2]
    v_d = g[3]
    dij_s = g[4]
    dji_d = g[5]
    y_s = g[6]
    tau_s = g[7]
    tsh_s = g[8]
    y_d = g[9]
    tau_d = g[10]
    tsh_d = g[11]

    sin_dij_s = jnp.sin(dij_s)
    a1 = th_s - th_d - dij_s
    p_from = (v_s * v_d * y_s / tau_s * jnp.sin(a1 - tsh_s)
              + (v_s / tau_s) ** 2 * y_s * sin_dij_s)
    p_to = (v_d * v_s * y_d / tau_d * jnp.sin(th_d - th_s - dji_d - tsh_d)
            + v_d ** 2 * y_d * jnp.sin(dji_d))
    msg = jnp.abs(
        v_s * v_d * y_s / tau_s * (jnp.sin(a1 - tsh_s)
                                   + jnp.sin(th_d - th_s - dij_s + tsh_s))
        + v_s / tau_s ** 2 * y_s * sin_dij_s
        + v_d ** 2 * y_s * sin_dij_s)

    o_ref[...] = jnp.stack([p_from, p_to])

    @pl.when(i == 0)
    def _():
        acc_ref[...] = jnp.zeros_like(acc_ref)
    acc_ref[...] += jnp.reshape(jnp.sum(msg), (1, 1))


def _physics(g):
    npad = PHY_N - g.shape[1]
    pad = jnp.zeros((12, npad), jnp.float32).at[7, :].set(1.0).at[10, :].set(1.0)
    g = jnp.concatenate([g, pad], axis=1)
    g = g.reshape(12, PHY_R, PHY_C)
    grid = PHY_R // PHY_RBLK
    out, msg_sum = pl.pallas_call(
        _physics_body,
        grid=(grid,),
        in_specs=[pl.BlockSpec((12, PHY_RBLK, PHY_C), lambda i: (0, i, 0))],
        out_specs=[pl.BlockSpec((2, PHY_RBLK, PHY_C), lambda i: (0, i, 0)),
                   pl.BlockSpec((1, 1), lambda i: (0, 0))],
        out_shape=[jax.ShapeDtypeStruct((2, PHY_R, PHY_C), jnp.float32),
                   jax.ShapeDtypeStruct((1, 1), jnp.float32)],
    )(g)
    out = out.reshape(2, PHY_N)[:, :N_EDGE]
    return out[0], out[1], msg_sum[0, 0]


def _edge_weights(params, k):
    blocks = [params['phi_v'], params['phi_theta'], params['phi_m']]
    w1 = jnp.concatenate([p['W1'][k] for p in blocks], axis=0).T
    b1 = jnp.concatenate([p['b1'][k] for p in blocks])[None, :]
    w2 = jax.scipy.linalg.block_diag(*[p['W2'][k].T for p in blocks])
    b2 = jnp.concatenate([p['b2'][k] for p in blocks])[None, :]
    w4 = jax.scipy.linalg.block_diag(*[p['W4'][k].T for p in blocks])
    b4 = jnp.concatenate([p['b4'][k] for p in blocks])[None, :]
    return w1, b1, w2, b2, w4, b4


def _bus_weights(params, k):
    blocks = [params['L_theta'], params['L_v'], params['L_m']]
    s_col = [4, 5, 6]
    w1s = []
    for bi, p in enumerate(blocks):
        w1t = p['W1'][k].T
        w = jnp.zeros((17, HIDDEN), jnp.float32)
        w = w.at[0:4, :].set(w1t[0:4, :])
        w = w.at[s_col[bi], :].set(w1t[4, :])
        w = w.at[7:17, :].set(w1t[5:15, :])
        w1s.append(w)
    w1 = jnp.concatenate(w1s, axis=1)
    b1 = jnp.concatenate([p['b1'][k] for p in blocks])[None, :]
    w2 = jax.scipy.linalg.block_diag(*[p['W2'][k].T for p in blocks])
    b2 = jnp.concatenate([p['b2'][k] for p in blocks])[None, :]
    w4 = jax.scipy.linalg.block_diag(*[p['W4'][k].T for p in blocks])
    b4 = jnp.concatenate([p['b4'][k] for p in blocks])[None, :]
    return w1, b1, w2, b2, w4, b4


@jax.jit
def _run(buses, lines, generators, params):
    src = lines[:, 0].astype(jnp.int32) - 1
    dst = lines[:, 1].astype(jnp.int32) - 1
    r = lines[:, 2]
    x_ = lines[:, 3]
    tau = lines[:, 4 + 1]
    tsh = lines[:, 6]
    y = 1.0 / jnp.sqrt(r ** 2 + x_ ** 2)
    y50 = y[:N_BUS]
    tau50 = tau[:N_BUS]
    tsh50 = tsh[:N_BUS]

    src2s = src[src]
    dst2s = dst[src]
    src2d = src[dst]
    dst2d = dst[dst]

    ys_e = y50[src]
    taus_e = tau50[src]
    tshs_e = tsh50[src]
    yd_e = y50[dst]
    taud_e = tau50[dst]
    tshd_e = tsh50[dst]
    static_cols = jnp.stack([ys_e, taus_e, tshs_e, yd_e, taud_e, tshd_e],
                            axis=0)

    pd = buses[:, 0]
    qd = buses[:, 1]
    gs = buses[:, 2]
    bs = buses[:, 3]

    gbus = generators[:, 0].astype(jnp.int32) - 1
    vg = generators[:, 3]
    pg0 = generators[:, 1]
    qg0 = generators[:, 2]
    pg_set = generators[:, 4]
    pmin = generators[:, 5]
    pmax = generators[:, 6]
    s_set = jnp.sum(pg_set)
    s_min = jnp.sum(pmin)
    s_max = jnp.sum(pmax)
    s_pd = jnp.sum(pd)

    m = jnp.zeros((N_BUS, LATENT), jnp.float32)
    v = jnp.ones((N_BUS,), jnp.float32).at[gbus].set(vg)
    theta = jnp.zeros((N_BUS,), jnp.float32)
    dp = jax.ops.segment_sum(pg0, gbus, num_segments=N_BUS) - pd - gs * v ** 2
    dq = jax.ops.segment_sum(qg0, gbus, num_segments=N_BUS) - qd + bs * v ** 2

    line_feats = lines[:, 2:7]
    total_loss = 0.0
    for k in range(K):
        e_in = jnp.concatenate([m[dst], line_feats], axis=1)
        ew = _edge_weights(params, k)
        e_out = _edge_mlp(e_in, *ew)
        seg = jax.ops.segment_sum(e_out, dst, num_segments=N_BUS)
        sv, st, sm = seg[:, 0], seg[:, 1], seg[:, 2]

        xb = jnp.concatenate(
            [jnp.stack([v, theta, dp, dq, st, sv, sm], axis=1), m], axis=1)
        bw = _bus_weights(params, k)
        b_out = _bus_mlp(xb, *bw)
        theta = b_out[:, 0]
        v = b_out[:, 1]
        m = b_out[:, 2:12]

        g = jnp.concatenate([
            jnp.stack([theta[src], theta[dst], v[src], v[dst],
                       theta[src2s] - theta[dst2s],
                       theta[dst2d] - theta[src2d]], axis=0),
            static_cols], axis=0)
        p_from, p_to, msg_sum = _physics(g)

        p_global = s_pd + jnp.sum(v ** 2 * gs) + msg_sum
        lam = jnp.where(p_global < s_set,
                        (p_global - s_min) / (2 * (s_set - s_min)),
                        (p_global - 2 * s_set + s_max) / (2 * (s_max - s_set)))
        pg_k = jnp.where(lam < 0.5,
                         pmin + 2 * (pg_set - pmin) * lam,
                         2 * pg_set - pmax + 2 * (pmax - pg_set) * lam)
        dp = (jax.ops.segment_sum(pg_k, gbus, num_segments=N_BUS)
              - pd - gs * v ** 2
              + jax.ops.segment_sum(p_from, dst, num_segments=N_BUS)
              + jax.ops.segment_sum(p_to, src, num_segments=N_BUS))
        dq = jnp.zeros_like(dp)
        total_loss = total_loss + GAMMA ** (K - k) * jnp.sum(dp ** 2)
    return total_loss


def kernel(buses, lines, generators, params):
    return _run(buses, lines, generators, params)


# speedup vs baseline: 4.8411x; 4.8411x over previous
import functools
import jax
import jax.numpy as jnp
from jax import lax
from jax.experimental import pallas as pl
from jax.experimental.pallas import tpu as pltpu
from jax.experimental.pallas import tpu_sc as plsc

N_BUS = 50000
N_BUSP = 50176
N_EDGE = 800000
LATENT = 10
HIDDEN = 10
K = 5
GAMMA = 0.9

NW = 32
CHUNK = 128
EPAD = 819200
EDGE_BLK = 8192
BUS_BLK = 6272
GPAD = 5120
ROWS_T = N_BUSP // 16

PHY_R = 1280
PHY_C = 640
PHY_RBLK = 128

_SLOPE = 0.01


def _leaky(x):
    return jnp.where(x >= 0, x, _SLOPE * x)


def _mesh():
    return plsc.VectorSubcoreMesh(core_axis_name="c", subcore_axis_name="s")


def _wid():
    return lax.axis_index("s") * 2 + lax.axis_index("c")


@functools.partial(
    pl.kernel, mesh=_mesh(),
    out_type=jax.ShapeDtypeStruct((LATENT * EPAD,), jnp.float32),
    scratch_types=[pltpu.VMEM((CHUNK,), jnp.int32),
                   pltpu.VMEM((CHUNK,), jnp.int32),
                   pltpu.VMEM((CHUNK,), jnp.float32),
                   pltpu.SemaphoreType.DMA])
def _sc_gather_m(mflat_hbm, idx_hbm, out_hbm, idx_v, idxc_v, buf_v, sem):
    base = _wid() * (EPAD // NW)

    def body(j, carry):
        off = base + j * CHUNK
        pltpu.sync_copy(idx_hbm.at[pl.ds(off, CHUNK)], idx_v)
        for c in range(LATENT):
            for u in range(CHUNK // 16):
                s = pl.ds(u * 16, 16)
                idxc_v[s] = idx_v[s] + c * N_BUSP
            pltpu.async_copy(mflat_hbm.at[idxc_v], buf_v, sem).wait()
            pltpu.sync_copy(buf_v, out_hbm.at[pl.ds(c * EPAD + off, CHUNK)])
        return carry

    lax.fori_loop(0, EPAD // NW // CHUNK, body, 0)


@functools.partial(
    pl.kernel, mesh=_mesh(),
    out_type=jax.ShapeDtypeStruct((6 * EPAD,), jnp.float32),
    scratch_types=[pltpu.VMEM((CHUNK,), jnp.int32),
                   pltpu.VMEM((CHUNK,), jnp.float32),
                   pltpu.VMEM((CHUNK,), jnp.float32),
                   pltpu.VMEM((CHUNK,), jnp.float32),
                   pltpu.SemaphoreType.DMA])
def _sc_gather_scalars(th_hbm, v_hbm, idx6_hbm, out_hbm,
                       idx_v, a_v, b_v, d_v, sem):
    base = _wid() * (EPAD // NW)

    def gat(table, idx_row, off, buf):
        pltpu.sync_copy(idx6_hbm.at[pl.ds(idx_row * EPAD + off, CHUNK)],
                        idx_v)
        pltpu.async_copy(table.at[idx_v], buf, sem).wait()

    def body(j, carry):
        off = base + j * CHUNK
        gat(th_hbm, 0, off, a_v)
        pltpu.sync_copy(a_v, out_hbm.at[pl.ds(0 * EPAD + off, CHUNK)])
        gat(th_hbm, 1, off, a_v)
        pltpu.sync_copy(a_v, out_hbm.at[pl.ds(1 * EPAD + off, CHUNK)])
        gat(v_hbm, 0, off, a_v)
        pltpu.sync_copy(a_v, out_hbm.at[pl.ds(2 * EPAD + off, CHUNK)])
        gat(v_hbm, 1, off, a_v)
        pltpu.sync_copy(a_v, out_hbm.at[pl.ds(3 * EPAD + off, CHUNK)])
        gat(th_hbm, 2, off, a_v)
        gat(th_hbm, 3, off, b_v)
        for u in range(CHUNK // 16):
            s = pl.ds(u * 16, 16)
            d_v[s] = a_v[s] - b_v[s]
        pltpu.sync_copy(d_v, out_hbm.at[pl.ds(4 * EPAD + off, CHUNK)])
        gat(th_hbm, 5, off, a_v)
        gat(th_hbm, 4, off, b_v)
        for u in range(CHUNK // 16):
            s = pl.ds(u * 16, 16)
            d_v[s] = a_v[s] - b_v[s]
        pltpu.sync_copy(d_v, out_hbm.at[pl.ds(5 * EPAD + off, CHUNK)])
        return carry

    lax.fori_loop(0, EPAD // NW // CHUNK, body, 0)


def _make_scatter(n_items, n_ch):
    per_w = n_items // NW
    chunk = CHUNK
    while per_w % chunk:
        chunk -= 8
    n_chunks = per_w // chunk

    scratch = ([pltpu.VMEM((chunk,), jnp.int32),
                pltpu.VMEM((chunk,), jnp.float32),
                pltpu.VMEM((ROWS_T,), jnp.float32)]
               + [pltpu.VMEM_SHARED((N_BUSP,), jnp.float32)
                  for _ in range(n_ch)]
               + [pltpu.SemaphoreType.DMA])

    @functools.partial(
        pl.kernel, mesh=_mesh(),
        out_type=jax.ShapeDtypeStruct((n_ch * 2 * N_BUSP,), jnp.float32),
        scratch_types=scratch)
    def k(vals_hbm, idx_hbm, zeros_hbm, out_hbm, idx_v, val_v, z_v,
          *rest):
        shareds = rest[:n_ch]
        sem = rest[n_ch]
        cid = lax.axis_index("c")
        sid = lax.axis_index("s")
        wid = sid * 2 + cid
        pltpu.sync_copy(zeros_hbm, z_v)
        for sh in shareds:
            pltpu.sync_copy(z_v, sh.at[pl.ds(sid * ROWS_T, ROWS_T)])
        plsc.subcore_barrier()

        base = wid * per_w

        def body(j, carry):
            off = base + j * chunk
            pltpu.sync_copy(idx_hbm.at[pl.ds(off, chunk)], idx_v)
            for c, sh in enumerate(shareds):
                pltpu.sync_copy(vals_hbm.at[pl.ds(c * n_items + off, chunk)],
                                val_v)
                pltpu.sync_copy(val_v, sh.at[idx_v], add=True)
            return carry

        lax.fori_loop(0, n_chunks, body, 0)
        plsc.subcore_barrier()
        for c, sh in enumerate(shareds):
            pltpu.sync_copy(sh.at[pl.ds(sid * ROWS_T, ROWS_T)], z_v)
            pltpu.sync_copy(
                z_v,
                out_hbm.at[pl.ds(c * 2 * N_BUSP + cid * N_BUSP
                                 + sid * ROWS_T, ROWS_T)])

    def run(vals_flat, idx, zeros):
        out = k(vals_flat, idx, zeros)
        return out.reshape(n_ch, 2, N_BUSP).sum(axis=1)

    return run


_scatter_e3 = _make_scatter(EPAD, 3)
_scatter_e1 = _make_scatter(EPAD, 1)
_scatter_g1 = _make_scatter(GPAD, 1)


def _edge_mlp_body(m_ref, lf_ref, w1m_ref, w1l_ref, b1_ref, w2_ref, b2_ref,
                   w4_ref, b4_ref, o_ref):
    i = pl.program_id(0)
    h = _leaky(jnp.dot(w1m_ref[...], m_ref[...],
                       preferred_element_type=jnp.float32)
               + jnp.dot(w1l_ref[...], lf_ref[...],
                         preferred_element_type=jnp.float32)
               + b1_ref[...])
    h = _leaky(jnp.dot(w2_ref[...], h, preferred_element_type=jnp.float32)
               + b2_ref[...])
    y = (jnp.dot(w4_ref[...], h, preferred_element_type=jnp.float32)
         + b4_ref[...])
    col = (i * EDGE_BLK
           + lax.broadcasted_iota(jnp.int32, (3, EDGE_BLK), 1))
    o_ref[...] = jnp.where(col < N_EDGE, y, 0.0)


def _edge_mlp(m10, lf, w1m, w1l, b1, w2, b2, w4, b4):
    grid = EPAD // EDGE_BLK
    full = lambda s: pl.BlockSpec(s, lambda i: (0, 0))
    return pl.pallas_call(
        _edge_mlp_body,
        grid=(grid,),
        in_specs=[
            pl.BlockSpec((LATENT, EDGE_BLK), lambda i: (0, i)),
            pl.BlockSpec((5, EDGE_BLK), lambda i: (0, i)),
            full(w1m.shape), full(w1l.shape), full(b1.shape),
            full(w2.shape), full(b2.shape), full(w4.shape), full(b4.shape),
        ],
        out_specs=pl.BlockSpec((3, EDGE_BLK), lambda i: (0, i)),
        out_shape=jax.ShapeDtypeStruct((3, EPAD), jnp.float32),
    )(m10, lf, w1m, w1l, b1, w2, b2, w4, b4)


def _bus_mlp_body(x_ref, w1_ref, b1_ref, w2_ref, b2_ref, w4_ref, b4_ref,
                  o_ref, acc_ref):
    i = pl.program_id(0)
    x = x_ref[...]
    h = _leaky(jnp.dot(w1_ref[...], x, preferred_element_type=jnp.float32)
               + b1_ref[...])
    h = _leaky(jnp.dot(w2_ref[...], h, preferred_element_type=jnp.float32)
               + b2_ref[...])
    y = (jnp.dot(w4_ref[...], h, preferred_element_type=jnp.float32)
         + b4_ref[...])
    theta_new = y[0:1, :] + x[1:2, :]
    v_new = y[1:2, :] + x[0:1, :]
    o_ref[...] = jnp.concatenate([theta_new, v_new, y[2:12, :]], axis=0)

    @pl.when(i == 0)
    def _():
        acc_ref[...] = jnp.zeros_like(acc_ref)
    col = (i * BUS_BLK
           + lax.broadcasted_iota(jnp.int32, (1, BUS_BLK), 1))
    vgs = jnp.where(col < N_BUS, v_new * v_new * x[17:18, :], 0.0)
    acc_ref[...] += jnp.reshape(jnp.sum(vgs), (1, 1))


def _bus_mlp(x, w1, b1, w2, b2, w4, b4):
    grid = N_BUSP // BUS_BLK
    full = lambda s: pl.BlockSpec(s, lambda i: (0, 0))
    return pl.pallas_call(
        _bus_mlp_body,
        grid=(grid,),
        in_specs=[
            pl.BlockSpec((18, BUS_BLK), lambda i: (0, i)),
            full(w1.shape), full(b1.shape), full(w2.shape), full(b2.shape),
            full(w4.shape), full(b4.shape),
        ],
        out_specs=[pl.BlockSpec((12, BUS_BLK), lambda i: (0, i)),
                   pl.BlockSpec((1, 1), lambda i: (0, 0))],
        out_shape=[jax.ShapeDtypeStruct((12, N_BUSP), jnp.float32),
                   jax.ShapeDtypeStruct((1, 1), jnp.float32)],
    )(x, w1, b1, w2, b2, w4, b4)


def _physics_body(gd_ref, gs_ref, o_ref, acc_ref):
    i = pl.program_id(0)
    gd = gd_ref[...]
    gsx = gs_ref[...]
    th_s = gd[0]
    th_d = gd[1]
    v_s = gd[2]
    v_d = gd[3]
    dij_s = gd[4]
    dji_d = gd[5]
    y_s = gsx[0]
    tau_s = gsx[1]
    tsh_s = gsx[2]
    y_d = gsx[3]
    tau_d = gsx[4]
    tsh_d = gsx[5]

    sin_dij_s = jnp.sin(dij_s)
    a1 = th_s - th_d - dij_s
    p_from = (v_s * v_d * y_s / tau_s * jnp.sin(a1 - tsh_s)
              + (v_s / tau_s) ** 2 * y_s * sin_dij_s)
    p_to = (v_d * v_s * y_d / tau_d * jnp.sin(th_d - th_s - dji_d - tsh_d)
            + v_d ** 2 * y_d * jnp.sin(dji_d))
    msg = jnp.abs(
        v_s * v_d * y_s / tau_s * (jnp.sin(a1 - tsh_s)
                                   + jnp.sin(th_d - th_s - dij_s + tsh_s))
        + v_s / tau_s ** 2 * y_s * sin_dij_s
        + v_d ** 2 * y_s * sin_dij_s)

    o_ref[...] = jnp.stack([p_from, p_to])

    @pl.when(i == 0)
    def _():
        acc_ref[...] = jnp.zeros_like(acc_ref)
    acc_ref[...] += jnp.reshape(jnp.sum(msg), (1, 1))


def _physics(g_dyn, g_static):
    g_dyn = g_dyn.reshape(6, PHY_R, PHY_C)
    g_static = g_static.reshape(6, PHY_R, PHY_C)
    grid = PHY_R // PHY_RBLK
    out, msg_sum = pl.pallas_call(
        _physics_body,
        grid=(grid,),
        in_specs=[pl.BlockSpec((6, PHY_RBLK, PHY_C), lambda i: (0, i, 0)),
                  pl.BlockSpec((6, PHY_RBLK, PHY_C), lambda i: (0, i, 0))],
        out_specs=[pl.BlockSpec((2, PHY_RBLK, PHY_C), lambda i: (0, i, 0)),
                   pl.BlockSpec((1, 1), lambda i: (0, 0))],
        out_shape=[jax.ShapeDtypeStruct((2, PHY_R, PHY_C), jnp.float32),
                   jax.ShapeDtypeStruct((1, 1), jnp.float32)],
    )(g_dyn, g_static)
    return out.reshape(2, EPAD), msg_sum[0, 0]


def _dp_body(pg_ref, pf_ref, pt_ref, v_ref, gp_ref, dp_ref, loss_ref):
    pgs = pg_ref[0] + pg_ref[1]
    pfs = pf_ref[0] + pf_ref[1]
    pts = pt_ref[0] + pt_ref[1]
    v = v_ref[...]
    gsx = gp_ref[0]
    pd = gp_ref[1]
    valid = gp_ref[2]
    dp = (pgs - pd - gsx * v * v + pfs + pts) * valid
    dp_ref[...] = dp
    loss_ref[...] = jnp.reshape(jnp.sum(dp * dp), (1, 1))


_DP_R = 392


def _dp_loss(pg_seg, pf_seg, pt_seg, v, gp):
    dp, loss = pl.pallas_call(
        _dp_body,
        in_specs=[pl.BlockSpec((2, _DP_R, 128), lambda: (0, 0, 0)),
                  pl.BlockSpec((2, _DP_R, 128), lambda: (0, 0, 0)),
                  pl.BlockSpec((2, _DP_R, 128), lambda: (0, 0, 0)),
                  pl.BlockSpec((_DP_R, 128), lambda: (0, 0)),
                  pl.BlockSpec((3, _DP_R, 128), lambda: (0, 0, 0))],
        out_specs=[pl.BlockSpec((_DP_R, 128), lambda: (0, 0)),
                   pl.BlockSpec((1, 1), lambda: (0, 0))],
        out_shape=[jax.ShapeDtypeStruct((_DP_R, 128), jnp.float32),
                   jax.ShapeDtypeStruct((1, 1), jnp.float32)],
    )(pg_seg.reshape(2, _DP_R, 128), pf_seg.reshape(2, _DP_R, 128),
      pt_seg.reshape(2, _DP_R, 128), v.reshape(_DP_R, 128),
      gp.reshape(3, _DP_R, 128))
    return dp.reshape(N_BUSP), loss[0, 0]


def _edge_weights(params, k):
    blocks = [params['phi_v'], params['phi_theta'], params['phi_m']]
    w1 = jnp.concatenate([p['W1'][k] for p in blocks], axis=0)
    w1m = w1[:, 0:LATENT]
    w1l = w1[:, LATENT:15]
    b1 = jnp.concatenate([p['b1'][k] for p in blocks])[:, None]
    w2 = jax.scipy.linalg.block_diag(*[p['W2'][k] for p in blocks])
    b2 = jnp.concatenate([p['b2'][k] for p in blocks])[:, None]
    w4 = jax.scipy.linalg.block_diag(*[p['W4'][k] for p in blocks])
    b4 = jnp.concatenate([p['b4'][k] for p in blocks])[:, None]
    return w1m, w1l, b1, w2, b2, w4, b4


def _bus_weights(params, k):
    blocks = [params['L_theta'], params['L_v'], params['L_m']]
    s_col = [4, 5, 6]
    w1s = []
    for bi, p in enumerate(blocks):
        w1k = p['W1'][k]
        w = jnp.zeros((HIDDEN, 18), jnp.float32)
        w = w.at[:, 0:4].set(w1k[:, 0:4])
        w = w.at[:, s_col[bi]].set(w1k[:, 4])
        w = w.at[:, 7:17].set(w1k[:, 5:15])
        w1s.append(w)
    w1 = jnp.concatenate(w1s, axis=0)
    b1 = jnp.concatenate([p['b1'][k] for p in blocks])[:, None]
    w2 = jax.scipy.linalg.block_diag(*[p['W2'][k] for p in blocks])
    b2 = jnp.concatenate([p['b2'][k] for p in blocks])[:, None]
    w4 = jax.scipy.linalg.block_diag(*[p['W4'][k] for p in blocks])
    b4 = jnp.concatenate([p['b4'][k] for p in blocks])[:, None]
    return w1, b1, w2, b2, w4, b4


def _pad_idx(idx):
    return jnp.concatenate(
        [idx, jnp.zeros((EPAD - N_EDGE,), jnp.int32)])


@jax.jit
def _run(buses, lines, generators, params):
    src = lines[:, 0].astype(jnp.int32) - 1
    dst = lines[:, 1].astype(jnp.int32) - 1
    r = lines[:, 2]
    x_ = lines[:, 3]
    tau = lines[:, 5]
    tsh = lines[:, 6]
    y = 1.0 / jnp.sqrt(r ** 2 + x_ ** 2)
    y50 = y[:N_BUS]
    tau50 = tau[:N_BUS]
    tsh50 = tsh[:N_BUS]

    src_pad = _pad_idx(src)
    dst_pad = _pad_idx(dst)
    idx6 = jnp.concatenate([
        src_pad, dst_pad,
        _pad_idx(src[src]), _pad_idx(dst[src]),
        _pad_idx(src[dst]), _pad_idx(dst[dst])])

    zpad = jnp.zeros((EPAD - N_EDGE,), jnp.float32)
    opad = jnp.ones((EPAD - N_EDGE,), jnp.float32)
    g_static = jnp.stack([
        jnp.concatenate([y50[src], zpad]),
        jnp.concatenate([tau50[src], opad]),
        jnp.concatenate([tsh50[src], zpad]),
        jnp.concatenate([y50[dst], zpad]),
        jnp.concatenate([tau50[dst], opad]),
        jnp.concatenate([tsh50[dst], zpad])])

    pd = buses[:, 0]
    qd = buses[:, 1]
    gs = buses[:, 2]
    bs = buses[:, 3]

    gbus = generators[:, 0].astype(jnp.int32) - 1
    vg = generators[:, 3]
    pg0 = generators[:, 1]
    qg0 = generators[:, 2]
    pg_set = generators[:, 4]
    pmin = generators[:, 5]
    pmax = generators[:, 6]
    s_set = jnp.sum(pg_set)
    s_min = jnp.sum(pmin)
    s_max = jnp.sum(pmax)
    s_pd = jnp.sum(pd)

    gbus_pad = jnp.concatenate(
        [gbus, jnp.zeros((GPAD - gbus.shape[0],), jnp.int32)])
    bus_zero_pad = jnp.zeros((N_BUSP - N_BUS,), jnp.float32)
    gp = jnp.stack([jnp.concatenate([gs, bus_zero_pad]),
                    jnp.concatenate([pd, bus_zero_pad]),
                    jnp.concatenate([jnp.ones((N_BUS,), jnp.float32),
                                     bus_zero_pad])])

    zeros_t = jnp.zeros((ROWS_T,), jnp.float32)
    lf_cm = jnp.concatenate(
        [lines[:, 2:7].T, jnp.zeros((5, EPAD - N_EDGE), jnp.float32)],
        axis=1)

    m_flat = jnp.zeros((LATENT * N_BUSP,), jnp.float32)
    v = jnp.ones((N_BUS,), jnp.float32).at[gbus].set(vg)
    theta = jnp.zeros((N_BUS,), jnp.float32)
    dp = jax.ops.segment_sum(pg0, gbus, num_segments=N_BUS) - pd - gs * v ** 2
    dq = jax.ops.segment_sum(qg0, gbus, num_segments=N_BUS) - qd + bs * v ** 2
    v_pad = jnp.concatenate([v, bus_zero_pad])
    theta_pad = jnp.concatenate([theta, bus_zero_pad])
    dp_pad = jnp.concatenate([dp, bus_zero_pad])
    dq_pad = jnp.concatenate([dq, bus_zero_pad])

    total_loss = 0.0
    for k in range(K):
        m10 = _sc_gather_m(m_flat, dst_pad).reshape(LATENT, EPAD)
        ew = _edge_weights(params, k)
        e_out = _edge_mlp(m10, lf_cm, *ew)
        seg = _scatter_e3(e_out.reshape(-1), dst_pad, zeros_t)
        sv, st, sm = seg[0], seg[1], seg[2]

        xb = jnp.concatenate(
            [jnp.stack([v_pad, theta_pad, dp_pad, dq_pad, st, sv, sm]),
             m_flat.reshape(LATENT, N_BUSP), gp[0:1]], axis=0)
        bw = _bus_weights(params, k)
        b_out, vgs_sum = _bus_mlp(xb, *bw)
        theta_pad = b_out[0]
        v_pad = b_out[1]
        m_flat = b_out[2:12].reshape(-1)
        vgs_sum = vgs_sum[0, 0]

        g_dyn = _sc_gather_scalars(theta_pad, v_pad, idx6).reshape(6, EPAD)
        pft, msg_sum = _physics(g_dyn, g_static)

        p_global = s_pd + vgs_sum + msg_sum
        lam = jnp.where(p_global < s_set,
                        (p_global - s_min) / (2 * (s_set - s_min)),
                        (p_global - 2 * s_set + s_max) / (2 * (s_max - s_set)))
        pg_k = jnp.where(lam < 0.5,
                         pmin + 2 * (pg_set - pmin) * lam,
                         2 * pg_set - pmax + 2 * (pmax - pg_set) * lam)
        pg_k_pad = jnp.concatenate(
            [pg_k, jnp.zeros((GPAD - pg_k.shape[0],), jnp.float32)])
        pg_seg = _scatter_g1(pg_k_pad, gbus_pad, zeros_t)
        pf_seg = _scatter_e1(pft[0], dst_pad, zeros_t)
        pt_seg = _scatter_e1(pft[1], src_pad, zeros_t)

        dp_pad, loss_k = _dp_loss_parts(pg_seg[0], pf_seg[0], pt_seg[0],
                                        v_pad, gp)
        dq_pad = jnp.zeros((N_BUSP,), jnp.float32)
        total_loss = total_loss + GAMMA ** (K - k) * loss_k
    return total_loss


def _dp_loss_parts(pg_sum, pf_sum, pt_sum, v, gp):
    z = jnp.zeros_like(pg_sum)
    return _dp_loss(jnp.stack([pg_sum, z]), jnp.stack([pf_sum, z]),
                    jnp.stack([pt_sum, z]), v, gp)


def kernel(buses, lines, generators, params):
    return _run(buses, lines, generators, params)
